# baseline (speedup 1.0000x reference)
import jax
import jax.numpy as jnp
from jax.experimental import pallas as pl
from jax.experimental.pallas import tpu as pltpu

N = 4096
B = 4
K = 25
RB = 512
NR = N // RB
C1 = 32
C2 = 64
N2 = N // 4
N4 = N // 16
FCB = 2048
NFCB = 16384 // FCB


def _split_kernel(l0_blk, hi_out):
    hi_out[...] = l0_blk[...].astype(jnp.bfloat16)


def _hl(v):
    hi = v.astype(jnp.bfloat16)
    lo = (v - hi.astype(jnp.float32)).astype(jnp.bfloat16)
    return jnp.concatenate([hi, lo], axis=1)


def _rep_blk(t, rows):
    return jnp.concatenate(
        [jnp.broadcast_to(t[:, b:b + 1], (rows, C1)) for b in range(B)],
        axis=1)


def _stage1_kernel(h_ref, xT, w1t, g1, be1, h1_out, tt, out_acc):
    x0 = xT[...]
    z = jnp.zeros((N, B), jnp.float32)
    tt[...] = jnp.concatenate([x0, z], axis=1).reshape(NR, RB, 2 * B)
    out_acc[...] = (_rep_blk(x0, N) * w1t[0:1, :]).reshape(NR, RB, B * C1)

    def chstep(src_a, w_row, first):
        so = 0 if src_a else B
        do = B if src_a else 0
        thl = _hl(tt[...].reshape(N, 2 * B)[:, so:so + B])

        def blk(j, _):
            m2 = jnp.dot(h_ref[j], thl,
                         preferred_element_type=jnp.float32)
            m = m2[:, :B] + m2[:, B:]
            tn = m if first else 2.0 * m - tt[j, :, do:do + B]
            tt[j, :, do:do + B] = tn
            out_acc[j] += _rep_blk(tn, RB) * w_row
            return _

        jax.lax.fori_loop(0, NR, blk, 0)

    chstep(True, w1t[1:2, :], True)
    chstep(False, w1t[2:3, :], False)

    def pair(i, _):
        chstep(True, w1t[pl.ds(2 * i + 1, 1), :], False)
        chstep(False, w1t[pl.ds(2 * i + 2, 1), :], False)
        return _

    jax.lax.fori_loop(1, 12, pair, 0)

    if True:
        o = out_acc[...].reshape(N, B * C1)
        s1f = jnp.sum(o, axis=0, keepdims=True)
        s2f = jnp.sum(o * o, axis=0, keepdims=True)
        s1 = sum(s1f[:, b * C1:(b + 1) * C1] for b in range(B))
        s2 = sum(s2f[:, b * C1:(b + 1) * C1] for b in range(B))
        cnt = float(N * B)
        mean = s1 / cnt
        var = s2 / cnt - mean * mean
        inv = jax.lax.rsqrt(var + 1e-5) * g1[...]
        shift = be1[...] - mean * inv
        invb = jnp.concatenate([inv] * B, axis=1)
        shiftb = jnp.concatenate([shift] * B, axis=1)
        y = jnp.maximum(o * invb + shiftb, 0.0)
        h1_out[...] = jnp.max(y.reshape(N2, 4, B * C1), axis=1)


def _stage2_kernel(l2_ref, h1, w2, g2, be2, h2_out, tbuf, out_acc):
    k = pl.program_id(0)
    s_cur = jax.lax.rem(k, 3)
    s_prev = jax.lax.rem(k + 2, 3)
    s_next = jax.lax.rem(k + 1, 3)

    @pl.when(k == 0)
    def _init():
        t0 = h1[...]
        tbuf[0] = t0
        w0 = w2[0]
        for b in range(B):
            out_acc[:, b * C2:(b + 1) * C2] = jnp.dot(
                t0[:, b * C1:(b + 1) * C1], w0,
                preferred_element_type=jnp.float32)

    t_cur = tbuf[s_cur]
    m = jnp.dot(l2_ref[...], t_cur, preferred_element_type=jnp.float32)
    t_next = jnp.where(k == 0, m, 2.0 * m - tbuf[s_prev])
    tbuf[s_next] = t_next

    wk = w2[k + 1]
    for b in range(B):
        out_acc[:, b * C2:(b + 1) * C2] += jnp.dot(
            t_next[:, b * C1:(b + 1) * C1], wk,
            preferred_element_type=jnp.float32)

    @pl.when(k == K - 2)
    def _epilogue():
        o = out_acc[...]
        s1f = jnp.sum(o, axis=0, keepdims=True)
        s2f = jnp.sum(o * o, axis=0, keepdims=True)
        s1 = sum(s1f[:, b * C2:(b + 1) * C2] for b in range(B))
        s2 = sum(s2f[:, b * C2:(b + 1) * C2] for b in range(B))
        cnt = float(N2 * B)
        mean = s1 / cnt
        var = s2 / cnt - mean * mean
        inv = jax.lax.rsqrt(var + 1e-5) * g2[...]
        shift = be2[...] - mean * inv
        invb = jnp.concatenate([inv] * B, axis=1)
        shiftb = jnp.concatenate([shift] * B, axis=1)
        y = jnp.maximum(o * invb + shiftb, 0.0)
        h2_out[...] = jnp.max(y.reshape(N4, 4, B * C2), axis=1)


def _fc_kernel(w1_blk, hflat_blk, fc1_b, fc2_w, fc2_b, out, acc):
    r = pl.program_id(0)

    @pl.when(r == 0)
    def _init():
        acc[...] = jnp.zeros((B, 512), jnp.float32)

    acc[...] += jnp.dot(hflat_blk[...], w1_blk[...],
                        preferred_element_type=jnp.float32)

    @pl.when(r == NFCB - 1)
    def _epilogue():
        z = jnp.maximum(acc[...] + fc1_b[0, :][None, :], 0.0)
        logits = jnp.dot(z, fc2_w[...],
                         preferred_element_type=jnp.float32) + fc2_b[0, :][None, :]
        mx = jnp.max(logits, axis=1, keepdims=True)
        sh = logits - mx
        lse = jnp.log(jnp.sum(jnp.exp(sh), axis=1, keepdims=True))
        out[...] = sh - lse


@jax.jit
def kernel(x, L0, L1, L2, W1, bb1, g1, be1, W2, bb2, g2, be2,
           fc1_w, fc1_b, fc2_w, fc2_b):
    xT = x.T
    w1 = W1[:, 0, :]
    del bb1, bb2, L1

    l0_hi = pl.pallas_call(
        _split_kernel,
        grid=(NR,),
        in_specs=[pl.BlockSpec((RB, N), lambda r: (r, 0))],
        out_specs=pl.BlockSpec((RB, N), lambda r: (r, 0)),
        out_shape=jax.ShapeDtypeStruct((N, N), jnp.bfloat16),
    )(L0)

    w1t = jnp.tile(w1, (1, B))
    h1 = pl.pallas_call(
        _stage1_kernel,
        out_shape=jax.ShapeDtypeStruct((N2, B * C1), jnp.float32),
        scratch_shapes=[
            pltpu.VMEM((NR, RB, 2 * B), jnp.float32),
            pltpu.VMEM((NR, RB, B * C1), jnp.float32),
        ],
    )(l0_hi.reshape(NR, RB, N), xT, w1t, g1.reshape(1, C1),
      be1.reshape(1, C1))

    h2 = pl.pallas_call(
        _stage2_kernel,
        grid=(K - 1,),
        in_specs=[
            pl.BlockSpec((N2, N2), lambda k: (0, 0)),
            pl.BlockSpec((N2, B * C1), lambda k: (0, 0)),
            pl.BlockSpec((K, C1, C2), lambda k: (0, 0, 0)),
            pl.BlockSpec((1, C2), lambda k: (0, 0)),
            pl.BlockSpec((1, C2), lambda k: (0, 0)),
        ],
        out_specs=pl.BlockSpec((N4, B * C2), lambda k: (0, 0)),
        out_shape=jax.ShapeDtypeStruct((N4, B * C2), jnp.float32),
        scratch_shapes=[
            pltpu.VMEM((3, N2, B * C1), jnp.float32),
            pltpu.VMEM((N2, B * C2), jnp.float32),
        ],
    )(L2, h1, W2, g2.reshape(1, C2), be2.reshape(1, C2))

    hflat = h2.reshape(N4, B, C2).transpose(1, 0, 2).reshape(B, N4 * C2)

    out = pl.pallas_call(
        _fc_kernel,
        grid=(NFCB,),
        in_specs=[
            pl.BlockSpec((FCB, 512), lambda r: (r, 0)),
            pl.BlockSpec((B, FCB), lambda r: (0, r)),
            pl.BlockSpec((1, 512), lambda r: (0, 0)),
            pl.BlockSpec((512, 10), lambda r: (0, 0)),
            pl.BlockSpec((1, 10), lambda r: (0, 0)),
        ],
        out_specs=pl.BlockSpec((B, 10), lambda r: (0, 0)),
        out_shape=jax.ShapeDtypeStruct((B, 10), jnp.float32),
        scratch_shapes=[
            pltpu.VMEM((B, 512), jnp.float32),
        ],
    )(fc1_w, hflat, fc1_b.reshape(1, 512), fc2_w, fc2_b.reshape(1, 10))

    return out


# speedup vs baseline: 1.2670x; 1.2670x over previous
import jax
import jax.numpy as jnp
from jax.experimental import pallas as pl
from jax.experimental.pallas import tpu as pltpu

N = 4096
B = 4
K = 25
RB = 512
NR = N // RB
C1 = 32
C2 = 64
N2 = N // 4
N4 = N // 16
FCB = 2048
NFCB = 16384 // FCB


def _split_kernel(l0_blk, hi_out):
    hi_out[...] = l0_blk[...].astype(jnp.bfloat16)


def _hl(v):
    hi = v.astype(jnp.bfloat16)
    lo = (v - hi.astype(jnp.float32)).astype(jnp.bfloat16)
    return jnp.concatenate([hi, lo], axis=1)


def _rep_blk(t, rows):
    return jnp.concatenate(
        [jnp.broadcast_to(t[:, b:b + 1], (rows, C1)) for b in range(B)],
        axis=1)


def _stage1_kernel(h_ref, xT, w1t, g1, be1, h1_out, tbuf, out_acc):
    x0 = xT[...]
    tbuf[...] = jnp.concatenate([_hl(x0), jnp.zeros((N, 2 * B), jnp.bfloat16)],
                                axis=1)
    out_acc[...] = _rep_blk(x0, N) * w1t[0:1, :]

    def step(k, _):
        k_even = jax.lax.rem(k, 2) == 0
        tb = tbuf[...]
        thl = jnp.where(k_even, tb[:, :2 * B], tb[:, 2 * B:])
        wk = w1t[pl.ds(k + 1, 1), :]

        for j in range(NR):
            rs = pl.ds(j * RB, RB)
            m2 = jnp.dot(h_ref[j * RB:(j + 1) * RB, :], thl,
                         preferred_element_type=jnp.float32)
            m = m2[:, :B] + m2[:, B:]
            tb_blk = tbuf[rs, :]
            p_hl = jnp.where(k_even, tb_blk[:, 2 * B:], tb_blk[:, :2 * B])
            c_hl = jnp.where(k_even, tb_blk[:, :2 * B], tb_blk[:, 2 * B:])
            t_prev_blk = (p_hl[:, :B].astype(jnp.float32)
                          + p_hl[:, B:].astype(jnp.float32))
            t_next_blk = jnp.where(k == 0, m, 2.0 * m - t_prev_blk)
            n_hl = _hl(t_next_blk)
            tbuf[rs, :] = jnp.where(
                k_even,
                jnp.concatenate([c_hl, n_hl], axis=1),
                jnp.concatenate([n_hl, c_hl], axis=1))
            out_acc[rs, :] += _rep_blk(t_next_blk, RB) * wk
        return _

    jax.lax.fori_loop(0, K - 1, step, 0)

    if True:
        o = out_acc[...]
        s1f = jnp.sum(o, axis=0, keepdims=True)
        s2f = jnp.sum(o * o, axis=0, keepdims=True)
        s1 = sum(s1f[:, b * C1:(b + 1) * C1] for b in range(B))
        s2 = sum(s2f[:, b * C1:(b + 1) * C1] for b in range(B))
        cnt = float(N * B)
        mean = s1 / cnt
        var = s2 / cnt - mean * mean
        inv = jax.lax.rsqrt(var + 1e-5) * g1[...]
        shift = be1[...] - mean * inv
        invb = jnp.concatenate([inv] * B, axis=1)
        shiftb = jnp.concatenate([shift] * B, axis=1)
        y = jnp.maximum(o * invb + shiftb, 0.0)
        h1_out[...] = jnp.max(y.reshape(N2, 4, B * C1), axis=1)


def _stage2_kernel(l2_ref, h1, w2, g2, be2, h2_out, tbuf, out_acc):
    k = pl.program_id(0)
    s_cur = jax.lax.rem(k, 3)
    s_prev = jax.lax.rem(k + 2, 3)
    s_next = jax.lax.rem(k + 1, 3)

    @pl.when(k == 0)
    def _init():
        t0 = h1[...]
        tbuf[0] = t0
        w0 = w2[0]
        for b in range(B):
            out_acc[:, b * C2:(b + 1) * C2] = jnp.dot(
                t0[:, b * C1:(b + 1) * C1], w0,
                preferred_element_type=jnp.float32)

    t_cur = tbuf[s_cur]
    m = jnp.dot(l2_ref[...], t_cur, preferred_element_type=jnp.float32)
    t_next = jnp.where(k == 0, m, 2.0 * m - tbuf[s_prev])
    tbuf[s_next] = t_next

    wk = w2[k + 1]
    for b in range(B):
        out_acc[:, b * C2:(b + 1) * C2] += jnp.dot(
            t_next[:, b * C1:(b + 1) * C1], wk,
            preferred_element_type=jnp.float32)

    @pl.when(k == K - 2)
    def _epilogue():
        o = out_acc[...]
        s1f = jnp.sum(o, axis=0, keepdims=True)
        s2f = jnp.sum(o * o, axis=0, keepdims=True)
        s1 = sum(s1f[:, b * C2:(b + 1) * C2] for b in range(B))
        s2 = sum(s2f[:, b * C2:(b + 1) * C2] for b in range(B))
        cnt = float(N2 * B)
        mean = s1 / cnt
        var = s2 / cnt - mean * mean
        inv = jax.lax.rsqrt(var + 1e-5) * g2[...]
        shift = be2[...] - mean * inv
        invb = jnp.concatenate([inv] * B, axis=1)
        shiftb = jnp.concatenate([shift] * B, axis=1)
        y = jnp.maximum(o * invb + shiftb, 0.0)
        h2_out[...] = jnp.max(y.reshape(N4, 4, B * C2), axis=1)


def _fc_kernel(w1_blk, hflat_blk, fc1_b, fc2_w, fc2_b, out, acc):
    r = pl.program_id(0)

    @pl.when(r == 0)
    def _init():
        acc[...] = jnp.zeros((B, 512), jnp.float32)

    acc[...] += jnp.dot(hflat_blk[...], w1_blk[...],
                        preferred_element_type=jnp.float32)

    @pl.when(r == NFCB - 1)
    def _epilogue():
        z = jnp.maximum(acc[...] + fc1_b[0, :][None, :], 0.0)
        logits = jnp.dot(z, fc2_w[...],
                         preferred_element_type=jnp.float32) + fc2_b[0, :][None, :]
        mx = jnp.max(logits, axis=1, keepdims=True)
        sh = logits - mx
        lse = jnp.log(jnp.sum(jnp.exp(sh), axis=1, keepdims=True))
        out[...] = sh - lse


@jax.jit
def kernel(x, L0, L1, L2, W1, bb1, g1, be1, W2, bb2, g2, be2,
           fc1_w, fc1_b, fc2_w, fc2_b):
    xT = x.T
    w1 = W1[:, 0, :]
    del bb1, bb2, L1

    l0_hi = pl.pallas_call(
        _split_kernel,
        grid=(NR,),
        in_specs=[pl.BlockSpec((RB, N), lambda r: (r, 0))],
        out_specs=pl.BlockSpec((RB, N), lambda r: (r, 0)),
        out_shape=jax.ShapeDtypeStruct((N, N), jnp.bfloat16),
    )(L0)

    w1t = jnp.tile(w1, (1, B))
    h1 = pl.pallas_call(
        _stage1_kernel,
        out_shape=jax.ShapeDtypeStruct((N2, B * C1), jnp.float32),
        scratch_shapes=[
            pltpu.VMEM((N, 4 * B), jnp.bfloat16),
            pltpu.VMEM((N, B * C1), jnp.float32),
        ],
    )(l0_hi, xT, w1t, g1.reshape(1, C1), be1.reshape(1, C1))

    h2 = pl.pallas_call(
        _stage2_kernel,
        grid=(K - 1,),
        in_specs=[
            pl.BlockSpec((N2, N2), lambda k: (0, 0)),
            pl.BlockSpec((N2, B * C1), lambda k: (0, 0)),
            pl.BlockSpec((K, C1, C2), lambda k: (0, 0, 0)),
            pl.BlockSpec((1, C2), lambda k: (0, 0)),
            pl.BlockSpec((1, C2), lambda k: (0, 0)),
        ],
        out_specs=pl.BlockSpec((N4, B * C2), lambda k: (0, 0)),
        out_shape=jax.ShapeDtypeStruct((N4, B * C2), jnp.float32),
        scratch_shapes=[
            pltpu.VMEM((3, N2, B * C1), jnp.float32),
            pltpu.VMEM((N2, B * C2), jnp.float32),
        ],
    )(L2, h1, W2, g2.reshape(1, C2), be2.reshape(1, C2))

    hflat = h2.reshape(N4, B, C2).transpose(1, 0, 2).reshape(B, N4 * C2)

    out = pl.pallas_call(
        _fc_kernel,
        grid=(NFCB,),
        in_specs=[
            pl.BlockSpec((FCB, 512), lambda r: (r, 0)),
            pl.BlockSpec((B, FCB), lambda r: (0, r)),
            pl.BlockSpec((1, 512), lambda r: (0, 0)),
            pl.BlockSpec((512, 10), lambda r: (0, 0)),
            pl.BlockSpec((1, 10), lambda r: (0, 0)),
        ],
        out_specs=pl.BlockSpec((B, 10), lambda r: (0, 0)),
        out_shape=jax.ShapeDtypeStruct((B, 10), jnp.float32),
        scratch_shapes=[
            pltpu.VMEM((B, 512), jnp.float32),
        ],
    )(fc1_w, hflat, fc1_b.reshape(1, 512), fc2_w, fc2_b.reshape(1, 10))

    return out
